# baseline (speedup 1.0000x reference)
import functools
import math

import jax
import jax.numpy as jnp
from jax import lax
from jax.experimental import pallas as pl
from jax.experimental.pallas import tpu as pltpu
from jax.experimental.pallas import tpu_sc as plsc

LANES = 16


def _make_sc_gather(n_flat, v_rows, d_embed, n_workers, chunk, off_len):
    mesh = plsc.VectorSubcoreMesh(core_axis_name="c", subcore_axis_name="s")
    per_w = n_flat // n_workers
    n_chunks = per_w // chunk

    @functools.partial(
        pl.kernel,
        mesh=mesh,
        compiler_params=pltpu.CompilerParams(use_tc_tiling_on_sc=False),
        out_type=jax.ShapeDtypeStruct((n_flat, d_embed), jnp.float32),
        scratch_types=[
            pltpu.VMEM((off_len,), jnp.int32),
            pltpu.VMEM((chunk,), jnp.int32),
            pltpu.VMEM((chunk, d_embed), jnp.float32),
            pltpu.SemaphoreType.DMA,
        ],
    )
    def k(idx_hbm, table_hbm, off_hbm, out_hbm, off_v, idx_v, rows_v, sem):
        nc = lax.axis_index("c")
        ns = lax.axis_index("s")
        wid = ns * 2 + nc
        pltpu.sync_copy(off_hbm, off_v)
        base0 = wid * per_w

        def body(ci, _):
            base = base0 + ci * chunk
            pltpu.sync_copy(idx_hbm.at[pl.ds(base, chunk)], idx_v)
            for j in range(chunk // off_len):
                for t in range(off_len // LANES):
                    p = j * off_len + t * LANES
                    idx_v[pl.ds(p, LANES)] = (
                        idx_v[pl.ds(p, LANES)] + off_v[pl.ds(t * LANES, LANES)]
                    )
            pltpu.async_copy(table_hbm.at[idx_v], rows_v, sem).wait()
            pltpu.sync_copy(rows_v, out_hbm.at[pl.ds(base, chunk)])
            return ()

        lax.fori_loop(0, n_chunks, body, (), unroll=False)

    return k


def kernel(x_cat, table, offsets):
    batch, ncol = x_cat.shape
    v_rows, d_embed = table.shape
    n_flat = batch * ncol
    n_workers = 32
    off_len = ncol * LANES // math.gcd(ncol, LANES)
    idx_flat = x_cat.reshape(n_flat).astype(jnp.int32)
    off_tiled = jnp.tile(offsets.astype(jnp.int32), off_len // ncol)
    chunk = 1664
    assert n_flat % (n_workers * chunk) == 0 and chunk % off_len == 0
    k = _make_sc_gather(n_flat, v_rows, d_embed, n_workers, chunk, off_len)
    out = k(idx_flat, table, off_tiled)
    return out.reshape(batch, ncol, d_embed)


# speedup vs baseline: 1.0035x; 1.0035x over previous
import functools
import math

import jax
import jax.numpy as jnp
from jax import lax
from jax.experimental import pallas as pl
from jax.experimental.pallas import tpu as pltpu
from jax.experimental.pallas import tpu_sc as plsc

LANES = 16


def _make_sc_gather(n_flat, d_embed, n_workers, chunk, off_len):
    mesh = plsc.VectorSubcoreMesh(core_axis_name="c", subcore_axis_name="s")
    per_w = n_flat // n_workers
    n_chunks = per_w // chunk

    @functools.partial(
        pl.kernel,
        mesh=mesh,
        compiler_params=pltpu.CompilerParams(use_tc_tiling_on_sc=False),
        out_type=jax.ShapeDtypeStruct((n_flat, d_embed), jnp.float32),
        scratch_types=[
            pltpu.VMEM((off_len,), jnp.int32),
            pltpu.VMEM((2, chunk), jnp.int32),
            pltpu.VMEM((2, chunk, d_embed), jnp.float32),
            pltpu.SemaphoreType.DMA,
            pltpu.SemaphoreType.DMA,
            pltpu.SemaphoreType.DMA,
            pltpu.SemaphoreType.DMA,
            pltpu.SemaphoreType.DMA,
        ],
    )
    def k(idx_hbm, table_hbm, off_hbm, out_hbm, off_v, idx_v, rows_v,
          sem_g, sem_i0, sem_i1, sem_o0, sem_o1):
        nc = lax.axis_index("c")
        ns = lax.axis_index("s")
        wid = ns * 2 + nc
        pltpu.sync_copy(off_hbm, off_v)
        base0 = wid * per_w
        sem_i = (sem_i0, sem_i1)
        sem_o = (sem_o0, sem_o1)

        def load(c):
            b = c % 2
            return pltpu.async_copy(
                idx_hbm.at[pl.ds(base0 + c * chunk, chunk)], idx_v.at[b],
                sem_i[b])

        def add_offsets(c):
            b = c % 2
            for j in range(chunk // off_len):
                for t in range(off_len // LANES):
                    p = j * off_len + t * LANES
                    idx_v[b, pl.ds(p, LANES)] = (
                        idx_v[b, pl.ds(p, LANES)]
                        + off_v[pl.ds(t * LANES, LANES)]
                    )

        def gather(c):
            b = c % 2
            return pltpu.async_copy(
                table_hbm.at[idx_v.at[b]], rows_v.at[b], sem_g)

        def store(c):
            b = c % 2
            return pltpu.async_copy(
                rows_v.at[b], out_hbm.at[pl.ds(base0 + c * chunk, chunk)],
                sem_o[b])

        cp_l0 = load(0)
        cp_l1 = load(1)
        cp_l0.wait()
        add_offsets(0)
        cp_g = gather(0)
        cp_s = [None, None]
        cp_l = [None, cp_l1]
        for c in range(n_chunks):
            b = c % 2
            nb = (c + 1) % 2
            if c + 1 < n_chunks:
                cp_l[nb].wait()
                add_offsets(c + 1)
            cp_g.wait()
            cp_s[b] = store(c)
            if c + 2 < n_chunks:
                cp_l[b] = load(c + 2)
            if c + 1 < n_chunks:
                if cp_s[nb] is not None:
                    cp_s[nb].wait()
                cp_g = gather(c + 1)
        if cp_s[(n_chunks - 2) % 2] is not None:
            cp_s[(n_chunks - 2) % 2].wait()
        cp_s[(n_chunks - 1) % 2].wait()

    return k


def kernel(x_cat, table, offsets):
    batch, ncol = x_cat.shape
    _, d_embed = table.shape
    n_flat = batch * ncol
    n_workers = 32
    off_len = ncol * LANES // math.gcd(ncol, LANES)
    idx_flat = x_cat.reshape(n_flat).astype(jnp.int32)
    off_tiled = jnp.tile(offsets.astype(jnp.int32), off_len // ncol)
    chunk = 1664
    assert n_flat % (n_workers * chunk) == 0 and chunk % off_len == 0
    k = _make_sc_gather(n_flat, d_embed, n_workers, chunk, off_len)
    out = k(idx_flat, table, off_tiled)
    return out.reshape(batch, ncol, d_embed)
